# baseline (speedup 1.0000x reference)
import functools

import jax
import jax.numpy as jnp
from jax import lax
from jax.experimental import pallas as pl
from jax.experimental.pallas import tpu as pltpu
from jax.experimental.pallas import tpu_sc as plsc

K = 8192
D = 256
N = 8192
BETA = 0.25

TB = 512
KB = 1024

NC = 2
NS = 16
NW = NC * NS
BPW = N // NW
CHUNK = 128
NCHUNK = BPW // CHUNK


def _dist_argmin_body(zn_ref, en_ref, z_ref, e_ref, idx_ref, dmin_ref,
                      run_min, run_idx):
    j = pl.program_id(1)

    @pl.when(j == 0)
    def _init():
        run_min[...] = jnp.full((TB, 1), jnp.inf, jnp.float32)
        run_idx[...] = jnp.zeros((TB, 1), jnp.int32)

    m = lax.dot_general(z_ref[...], e_ref[...],
                        (((1,), (1,)), ((), ())),
                        preferred_element_type=jnp.float32)
    d = (zn_ref[...] + en_ref[...]) - 2.0 * m

    lmin = jnp.min(d, axis=1, keepdims=True)
    col = lax.broadcasted_iota(jnp.int32, (TB, KB), 1) + j * KB
    larg = jnp.min(jnp.where(d == lmin, col, jnp.int32(2 ** 30)),
                   axis=1, keepdims=True)

    better = lmin < run_min[...]
    run_min[...] = jnp.where(better, lmin, run_min[...])
    run_idx[...] = jnp.where(better, larg, run_idx[...])

    @pl.when(j == pl.num_programs(1) - 1)
    def _emit():
        idx_ref[...] = run_idx[...]
        dmin_ref[...] = run_min[...]


def _dist_argmin(z_flat, embedding, zn, en):
    grid = (N // TB, K // KB)
    return pl.pallas_call(
        _dist_argmin_body,
        grid=grid,
        in_specs=[
            pl.BlockSpec((TB, 1), lambda i, j: (i, 0)),
            pl.BlockSpec((1, KB), lambda i, j: (0, j)),
            pl.BlockSpec((TB, D), lambda i, j: (i, 0)),
            pl.BlockSpec((KB, D), lambda i, j: (j, 0)),
        ],
        out_specs=[
            pl.BlockSpec((TB, 1), lambda i, j: (i, 0)),
            pl.BlockSpec((TB, 1), lambda i, j: (i, 0)),
        ],
        out_shape=[
            jax.ShapeDtypeStruct((N, 1), jnp.int32),
            jax.ShapeDtypeStruct((N, 1), jnp.float32),
        ],
        scratch_shapes=[
            pltpu.VMEM((TB, 1), jnp.float32),
            pltpu.VMEM((TB, 1), jnp.int32),
        ],
        compiler_params=pltpu.CompilerParams(
            dimension_semantics=("arbitrary", "arbitrary")),
    )(zn, en, z_flat, embedding)


def _gather_hist_body(emb_hbm, idx_hbm, out_hbm, hist_hbm,
                      idx_v, rows_v, hist_v, sem):
    wid = lax.axis_index("s") * NC + lax.axis_index("c")

    pltpu.sync_copy(idx_hbm.at[wid], idx_v)

    cps = [pltpu.async_copy(emb_hbm.at[idx_v.at[c]], rows_v.at[c], sem)
           for c in range(NCHUNK)]
    for cp in cps:
        cp.wait()

    def _zero(i, carry):
        hist_v[pl.ds(i * 16, 16)] = jnp.zeros((16,), jnp.float32)
        return carry
    lax.fori_loop(0, K // 16, _zero, 0)

    ones = jnp.ones((16,), jnp.float32)
    for c in range(NCHUNK):
        def _acc(g, carry):
            iv = idx_v[c, pl.ds(g * 16, 16)]
            plsc.addupdate_scatter(hist_v, [iv], ones)
            return carry
        lax.fori_loop(0, CHUNK // 16, _acc, 0)

    pltpu.sync_copy(rows_v, out_hbm.at[pl.ds(wid * NCHUNK, NCHUNK)])
    pltpu.sync_copy(hist_v, hist_hbm.at[wid])


def _gather_hist(embedding, idx3):
    mesh = plsc.VectorSubcoreMesh(core_axis_name="c", subcore_axis_name="s")
    fn = pl.kernel(
        _gather_hist_body,
        mesh=mesh,
        out_type=[
            jax.ShapeDtypeStruct((N // CHUNK, CHUNK, D), jnp.float32),
            jax.ShapeDtypeStruct((NW, K), jnp.float32),
        ],
        scratch_types=[
            pltpu.VMEM((NCHUNK, CHUNK), jnp.int32),
            pltpu.VMEM((NCHUNK, CHUNK, D), jnp.float32),
            pltpu.VMEM((K,), jnp.float32),
            pltpu.SemaphoreType.DMA,
        ],
        compiler_params=pltpu.CompilerParams(needs_layout_passes=False),
    )
    return fn(embedding, idx3)


def _losses_body(hist_ref, dmin_ref, loss_ref, perp_ref):
    counts = jnp.sum(hist_ref[...], axis=0, keepdims=True)
    p = counts * jnp.float32(1.0 / N)
    ent = jnp.sum(p * jnp.log(p + 1e-10))
    perp_ref[0, 0] = jnp.exp(-ent)
    loss_ref[0, 0] = (BETA / (N * D)) * jnp.sum(dmin_ref[...])


def _losses(hist, dmin):
    return pl.pallas_call(
        _losses_body,
        out_specs=[
            pl.BlockSpec(memory_space=pltpu.SMEM),
            pl.BlockSpec(memory_space=pltpu.SMEM),
        ],
        out_shape=[
            jax.ShapeDtypeStruct((1, 1), jnp.float32),
            jax.ShapeDtypeStruct((1, 1), jnp.float32),
        ],
    )(hist, dmin.reshape(N // 128, 128))


def kernel(z, embedding):
    z_flat = jnp.reshape(z, (-1, D))
    zn = jnp.sum(z_flat ** 2, axis=1, keepdims=True)
    en = jnp.sum(embedding ** 2, axis=1)[None, :]

    idx2d, dmin = _dist_argmin(z_flat, embedding, zn, en)
    encoding_indices = idx2d.reshape(N)

    quantized3, hist = _gather_hist(
        embedding, idx2d.reshape(NW, NCHUNK, CHUNK))
    quantized_st = quantized3.reshape(z.shape)

    loss2d, perp2d = _losses(hist, dmin)

    return (quantized_st, encoding_indices, jnp.zeros(()),
            loss2d.reshape(()), perp2d.reshape(()))


# speedup vs baseline: 1.2265x; 1.2265x over previous
import functools

import jax
import jax.numpy as jnp
from jax import lax
from jax.experimental import pallas as pl
from jax.experimental.pallas import tpu as pltpu
from jax.experimental.pallas import tpu_sc as plsc

K = 8192
D = 256
N = 8192
BETA = 0.25

TB = 512
KB = 2048

NC = 2
NS = 16
NW = NC * NS
BPW = N // NW
CHUNK = 128
NCHUNK = BPW // CHUNK


def _dist_argmin_body(zn_ref, en_ref, z_ref, e_ref, idx_ref, dmin_ref,
                      run_min, run_idx):
    j = pl.program_id(1)

    @pl.when(j == 0)
    def _init():
        run_min[...] = jnp.full((TB, 1), jnp.inf, jnp.float32)
        run_idx[...] = jnp.zeros((TB, 1), jnp.float32)

    m = lax.dot_general(z_ref[...], e_ref[...],
                        (((1,), (1,)), ((), ())),
                        preferred_element_type=jnp.float32)
    d = (zn_ref[...] + en_ref[...]) + m

    lmin = jnp.min(d, axis=1, keepdims=True)
    col = lax.broadcasted_iota(jnp.int32, (1, KB), 1).astype(jnp.float32)
    larg = jnp.min(jnp.where(d == lmin, col, jnp.float32(jnp.inf)),
                   axis=1, keepdims=True) + jnp.float32(j * KB)

    better = lmin < run_min[...]
    run_min[...] = jnp.where(better, lmin, run_min[...])
    run_idx[...] = jnp.where(better, larg, run_idx[...])

    @pl.when(j == pl.num_programs(1) - 1)
    def _emit():
        idx_ref[...] = run_idx[...].astype(jnp.int32)
        dmin_ref[...] = run_min[...]


def _dist_argmin(z_flat, embedding, zn, en):
    grid = (N // TB, K // KB)
    return pl.pallas_call(
        _dist_argmin_body,
        grid=grid,
        in_specs=[
            pl.BlockSpec((TB, 1), lambda i, j: (i, 0)),
            pl.BlockSpec((1, KB), lambda i, j: (0, j)),
            pl.BlockSpec((TB, D), lambda i, j: (i, 0)),
            pl.BlockSpec((KB, D), lambda i, j: (j, 0)),
        ],
        out_specs=[
            pl.BlockSpec((TB, 1), lambda i, j: (i, 0)),
            pl.BlockSpec((TB, 1), lambda i, j: (i, 0)),
        ],
        out_shape=[
            jax.ShapeDtypeStruct((N, 1), jnp.int32),
            jax.ShapeDtypeStruct((N, 1), jnp.float32),
        ],
        scratch_shapes=[
            pltpu.VMEM((TB, 1), jnp.float32),
            pltpu.VMEM((TB, 1), jnp.float32),
        ],
        compiler_params=pltpu.CompilerParams(
            dimension_semantics=("arbitrary", "arbitrary")),
    )(zn, en, z_flat, embedding)


def _gather_hist_body(emb_hbm, idx_hbm, out_hbm, hist_hbm,
                      idx_v, rows_v, hist_v, sem):
    wid = lax.axis_index("s") * NC + lax.axis_index("c")

    pltpu.sync_copy(idx_hbm.at[wid], idx_v)

    cps = [pltpu.async_copy(emb_hbm.at[idx_v.at[c]], rows_v.at[c], sem)
           for c in range(NCHUNK)]
    for cp in cps:
        cp.wait()

    def _zero(i, carry):
        hist_v[pl.ds(i * 16, 16)] = jnp.zeros((16,), jnp.float32)
        return carry
    lax.fori_loop(0, K // 16, _zero, 0)

    ones = jnp.ones((16,), jnp.float32)
    for c in range(NCHUNK):
        def _acc(g, carry):
            iv = idx_v[c, pl.ds(g * 16, 16)]
            plsc.addupdate_scatter(hist_v, [iv], ones)
            return carry
        lax.fori_loop(0, CHUNK // 16, _acc, 0)

    pltpu.sync_copy(rows_v, out_hbm.at[pl.ds(wid * NCHUNK, NCHUNK)])
    pltpu.sync_copy(hist_v, hist_hbm.at[wid])


def _gather_hist(embedding, idx3):
    mesh = plsc.VectorSubcoreMesh(core_axis_name="c", subcore_axis_name="s")
    fn = pl.kernel(
        _gather_hist_body,
        mesh=mesh,
        out_type=[
            jax.ShapeDtypeStruct((N // CHUNK, CHUNK, D), jnp.float32),
            jax.ShapeDtypeStruct((NW, K), jnp.float32),
        ],
        scratch_types=[
            pltpu.VMEM((NCHUNK, CHUNK), jnp.int32),
            pltpu.VMEM((NCHUNK, CHUNK, D), jnp.float32),
            pltpu.VMEM((K,), jnp.float32),
            pltpu.SemaphoreType.DMA,
        ],
        compiler_params=pltpu.CompilerParams(needs_layout_passes=False),
    )
    return fn(embedding, idx3)


def _losses_body(hist_ref, dmin_ref, loss_ref, perp_ref):
    counts = jnp.sum(hist_ref[...], axis=0, keepdims=True)
    p = counts * jnp.float32(1.0 / N)
    ent = jnp.sum(p * jnp.log(p + 1e-10))
    perp_ref[0, 0] = jnp.exp(-ent)
    loss_ref[0, 0] = (BETA / (N * D)) * jnp.sum(dmin_ref[...])


def _losses(hist, dmin):
    return pl.pallas_call(
        _losses_body,
        out_specs=[
            pl.BlockSpec(memory_space=pltpu.SMEM),
            pl.BlockSpec(memory_space=pltpu.SMEM),
        ],
        out_shape=[
            jax.ShapeDtypeStruct((1, 1), jnp.float32),
            jax.ShapeDtypeStruct((1, 1), jnp.float32),
        ],
    )(hist, dmin.reshape(N // 128, 128))


def kernel(z, embedding):
    z_flat = jnp.reshape(z, (-1, D))
    zn = jnp.sum(z_flat ** 2, axis=1, keepdims=True)
    en = jnp.sum(embedding ** 2, axis=1)[None, :]

    idx2d, dmin = _dist_argmin(z_flat * jnp.float32(-2.0), embedding, zn, en)
    encoding_indices = idx2d.reshape(N)

    quantized3, hist = _gather_hist(
        embedding, idx2d.reshape(NW, NCHUNK, CHUNK))
    quantized_st = quantized3.reshape(z.shape)

    loss2d, perp2d = _losses(hist, dmin)

    return (quantized_st, encoding_indices, jnp.zeros(()),
            loss2d.reshape(()), perp2d.reshape(()))
